# Initial kernel scaffold; baseline (speedup 1.0000x reference)
import jax
import jax.numpy as jnp
from jax.experimental import pallas as pl


def kernel(rule_features, selected_mask, expert_bias, selected_idx):
    raise NotImplementedError("write your pallas kernel here")


# speedup vs baseline: 1.8334x; 1.8334x over previous
import functools
import math
import struct

import jax
import jax.numpy as jnp
from jax import lax
from jax.experimental import pallas as pl
from jax.experimental.pallas import tpu as pltpu
from jax.experimental.pallas import tpu_sc as plsc

N_TOK = 16384
N_FEAT = 64
N_EXPERTS = 16
N_SEL = 4
N_BINS = 5

_NC = 2
_NS = 16
_NW = _NC * _NS
_CHUNK = N_TOK // _NW


def _fbits(i):
    return struct.unpack('<f', struct.pack('<I', i))[0]


_ERF_T = [_fbits(0xBF57747E), _fbits(0xBE81B6B6),
          _fbits(0x3E81B6B6), _fbits(0x3F57747F)]
_CLAMP_T = [_fbits(0x3E4CCCCD), _fbits(0x3ECCCCCD),
            _fbits(0x3F19999A), _fbits(0x3F4CCCCD)]
_BC = [_fbits(0x3DCCCCCD), _fbits(0x3E99999A), _fbits(0x3F000000),
       _fbits(0x3F333333), _fbits(0x3F666667)]

_TC_BLK = 2048


def _tc_minmax_body(cm_ref, x_ref, out_ref, mn_ref, mx_ref):
    i = pl.program_id(0)
    x = x_ref[...]
    cm = cm_ref[...] > 0.0
    big = jnp.float32(3.4028235e38)
    mn = jnp.min(jnp.where(cm, x, big))
    mx = jnp.max(jnp.where(cm, x, -big))

    @pl.when(i == 0)
    def _():
        mn_ref[0] = mn
        mx_ref[0] = mx

    @pl.when(i > 0)
    def _():
        mn_ref[0] = jnp.minimum(mn_ref[0], mn)
        mx_ref[0] = jnp.maximum(mx_ref[0], mx)

    @pl.when(i == pl.num_programs(0) - 1)
    def _():
        lo = mn_ref[0]
        hi = mx_ref[0]
        already_ratio = jnp.logical_and(lo >= -1e-06, hi <= 1.0 + 1e-06)
        row = lax.broadcasted_iota(jnp.int32, (8, 128), 0)
        t = jnp.zeros((8, 128), jnp.float32)
        for j in range(4):
            tj = jnp.where(already_ratio, jnp.float32(_CLAMP_T[j]),
                           jnp.float32(_ERF_T[j]))
            t = jnp.where(row == j, tj, t)
        out_ref[...] = t


def _thresholds_tc(rule_features, colmask2d):
    return pl.pallas_call(
        _tc_minmax_body,
        out_shape=jax.ShapeDtypeStruct((8, 128), jnp.float32),
        grid=(N_TOK // _TC_BLK,),
        in_specs=[
            pl.BlockSpec((1, N_FEAT), lambda i: (0, 0)),
            pl.BlockSpec((_TC_BLK, N_FEAT), lambda i: (i, 0)),
        ],
        out_specs=pl.BlockSpec((8, 128), lambda i: (0, 0)),
        scratch_shapes=[
            pltpu.SMEM((1,), jnp.float32),
            pltpu.SMEM((1,), jnp.float32),
        ],
        compiler_params=pltpu.CompilerParams(
            dimension_semantics=("arbitrary",)),
    )(colmask2d, rule_features)


def _sc_router_body(feat, maskf, bias, idxf, thr,
                    w_out, l_out,
                    fbuf, wbuf, lbuf, mbuf, bbuf, ibuf, tbuf):
    wid = lax.axis_index("s") * _NC + lax.axis_index("c")
    base = wid * _CHUNK

    pltpu.sync_copy(feat.at[pl.ds(base * N_FEAT, _CHUNK * N_FEAT)], fbuf)
    pltpu.sync_copy(maskf, mbuf)
    pltpu.sync_copy(bias, bbuf)
    pltpu.sync_copy(idxf, ibuf)
    pltpu.sync_copy(thr, tbuf)

    lanes = lax.iota(jnp.int32, 16)
    four = jnp.full((16,), 4, jnp.int32)
    m_cols = [plsc.load_gather(mbuf, [lanes * four + s]) for s in range(4)]
    i_cols = [plsc.load_gather(ibuf, [lanes * four + s]) for s in range(4)]
    bias_v = bbuf[...]
    cnt = jnp.maximum((m_cols[0] + m_cols[2]) + (m_cols[1] + m_cols[3]),
                      jnp.full((16,), jnp.float32(1.0)))
    thr_v = [tbuf[pl.ds(128 * j, 16)] for j in range(4)]
    bc_v = [jnp.full((16,), jnp.float32(c)) for c in _BC]
    neg_big = jnp.full((16,), jnp.float32(-3.0e38))
    one_v = jnp.full((16,), jnp.float32(1.0))
    zero_v = jnp.full((16,), jnp.float32(0.0))

    def body(t, carry):
        row = jnp.full((16,), t * N_FEAT, jnp.int32)
        p = []
        for s in range(4):
            g = plsc.load_gather(fbuf, [row + i_cols[s]])
            bc = bc_v[0]
            bc = jnp.where(g >= thr_v[0], bc_v[1], bc)
            bc = jnp.where(g >= thr_v[1], bc_v[2], bc)
            bc = jnp.where(g >= thr_v[2], bc_v[3], bc)
            bc = jnp.where(g >= thr_v[3], bc_v[4], bc)
            p.append(bc * m_cols[s])
        ssum = (p[0] + p[2]) + (p[1] + p[3])
        logits = ssum / cnt + bias_v
        m1 = jnp.max(logits)
        oh1 = lanes == plsc.all_reduce_ffs(logits == m1)
        l2 = jnp.where(oh1, neg_big, logits)
        m2 = jnp.max(l2)
        oh2 = lanes == plsc.all_reduce_ffs(l2 == m2)
        u = jnp.exp(jnp.full((16,), m2 - m1))
        den = one_v + u
        w1 = one_v / den
        w2 = u / den
        wv = jnp.where(oh1, w1, jnp.where(oh2, w2, zero_v))
        lbuf[pl.ds(t * N_EXPERTS, N_EXPERTS)] = logits
        wbuf[pl.ds(t * N_EXPERTS, N_EXPERTS)] = wv
        return carry

    lax.fori_loop(0, _CHUNK, body, 0)

    pltpu.sync_copy(wbuf, w_out.at[pl.ds(base * N_EXPERTS, _CHUNK * N_EXPERTS)])
    pltpu.sync_copy(lbuf, l_out.at[pl.ds(base * N_EXPERTS, _CHUNK * N_EXPERTS)])


def _make_sc_router():
    return pl.kernel(
        _sc_router_body,
        out_type=[
            jax.ShapeDtypeStruct((N_TOK * N_EXPERTS,), jnp.float32),
            jax.ShapeDtypeStruct((N_TOK * N_EXPERTS,), jnp.float32),
        ],
        mesh=plsc.VectorSubcoreMesh(
            core_axis_name="c", subcore_axis_name="s",
            num_cores=_NC, num_subcores=_NS),
        scratch_types=[
            pltpu.VMEM((_CHUNK * N_FEAT,), jnp.float32),
            pltpu.VMEM((_CHUNK * N_EXPERTS,), jnp.float32),
            pltpu.VMEM((_CHUNK * N_EXPERTS,), jnp.float32),
            pltpu.VMEM((N_EXPERTS * N_SEL,), jnp.float32),
            pltpu.VMEM((N_EXPERTS,), jnp.float32),
            pltpu.VMEM((N_EXPERTS * N_SEL,), jnp.int32),
            pltpu.VMEM((1024,), jnp.float32),
        ],
        compiler_params=pltpu.CompilerParams(needs_layout_passes=False),
    )


def kernel(rule_features, selected_mask, expert_bias, selected_idx):
    idx_flat = selected_idx.reshape(-1).astype(jnp.int32)
    colmask = jnp.zeros((N_FEAT,), jnp.float32).at[idx_flat].set(1.0)
    thr = _thresholds_tc(rule_features, colmask.reshape(1, N_FEAT))
    w_flat, l_flat = _make_sc_router()(
        rule_features.reshape(-1),
        selected_mask.reshape(-1).astype(jnp.float32),
        expert_bias.astype(jnp.float32),
        idx_flat,
        thr.reshape(-1),
    )
    weights = w_flat.reshape(N_TOK, N_EXPERTS)
    scaled_logits = l_flat.reshape(N_TOK, N_EXPERTS)
    return (weights, scaled_logits)


# speedup vs baseline: 2.2057x; 1.2030x over previous
import functools
import math
import struct

import jax
import jax.numpy as jnp
from jax import lax
from jax.experimental import pallas as pl
from jax.experimental.pallas import tpu as pltpu
from jax.experimental.pallas import tpu_sc as plsc

N_TOK = 16384
N_FEAT = 64
N_EXPERTS = 16
N_SEL = 4
N_BINS = 5

_NC = 2
_NS = 16
_NW = _NC * _NS
_CHUNK = N_TOK // _NW


def _fbits(i):
    return struct.unpack('<f', struct.pack('<I', i))[0]


_ERF_T = [_fbits(0xBF57747E), _fbits(0xBE81B6B6),
          _fbits(0x3E81B6B6), _fbits(0x3F57747F)]
_CLAMP_T = [_fbits(0x3E4CCCCD), _fbits(0x3ECCCCCD),
            _fbits(0x3F19999A), _fbits(0x3F4CCCCD)]
_BC = [_fbits(0x3DCCCCCD), _fbits(0x3E99999A), _fbits(0x3F000000),
       _fbits(0x3F333333), _fbits(0x3F666667)]

_TC_BLK = 2048


def _tc_minmax_body(cm_ref, x_ref, out_ref, mn_ref, mx_ref):
    i = pl.program_id(0)
    x = x_ref[...]
    cm = cm_ref[...] > 0.0
    big = jnp.float32(3.4028235e38)
    mn = jnp.min(jnp.where(cm, x, big))
    mx = jnp.max(jnp.where(cm, x, -big))

    @pl.when(i == 0)
    def _():
        mn_ref[0] = mn
        mx_ref[0] = mx

    @pl.when(i > 0)
    def _():
        mn_ref[0] = jnp.minimum(mn_ref[0], mn)
        mx_ref[0] = jnp.maximum(mx_ref[0], mx)

    @pl.when(i == pl.num_programs(0) - 1)
    def _():
        lo = mn_ref[0]
        hi = mx_ref[0]
        already_ratio = jnp.logical_and(lo >= -1e-06, hi <= 1.0 + 1e-06)
        row = lax.broadcasted_iota(jnp.int32, (8, 128), 0)
        t = jnp.zeros((8, 128), jnp.float32)
        for j in range(4):
            tj = jnp.where(already_ratio, jnp.float32(_CLAMP_T[j]),
                           jnp.float32(_ERF_T[j]))
            t = jnp.where(row == j, tj, t)
        out_ref[...] = t


def _thresholds_tc(rule_features, colmask2d):
    return pl.pallas_call(
        _tc_minmax_body,
        out_shape=jax.ShapeDtypeStruct((8, 128), jnp.float32),
        grid=(N_TOK // _TC_BLK,),
        in_specs=[
            pl.BlockSpec((1, N_FEAT), lambda i: (0, 0)),
            pl.BlockSpec((_TC_BLK, N_FEAT), lambda i: (i, 0)),
        ],
        out_specs=pl.BlockSpec((8, 128), lambda i: (0, 0)),
        scratch_shapes=[
            pltpu.SMEM((1,), jnp.float32),
            pltpu.SMEM((1,), jnp.float32),
        ],
        compiler_params=pltpu.CompilerParams(
            dimension_semantics=("arbitrary",)),
    )(colmask2d, rule_features)


def _sc_router_body(feat, maskf, bias, idxf, thr,
                    w_out, l_out,
                    fbuf, wbuf, lbuf, mbuf, bbuf, ibuf, tbuf):
    wid = lax.axis_index("s") * _NC + lax.axis_index("c")
    base = wid * _CHUNK

    pltpu.sync_copy(feat.at[pl.ds(base * N_FEAT, _CHUNK * N_FEAT)], fbuf)
    pltpu.sync_copy(maskf, mbuf)
    pltpu.sync_copy(bias, bbuf)
    pltpu.sync_copy(idxf, ibuf)
    pltpu.sync_copy(thr, tbuf)

    lanes = lax.iota(jnp.int32, 16)
    four = jnp.full((16,), 4, jnp.int32)
    m_cols = [plsc.load_gather(mbuf, [lanes * four + s]) for s in range(4)]
    i_cols = [plsc.load_gather(ibuf, [lanes * four + s]) for s in range(4)]
    bias_v = bbuf[...]
    cnt = jnp.maximum((m_cols[0] + m_cols[2]) + (m_cols[1] + m_cols[3]),
                      jnp.full((16,), jnp.float32(1.0)))
    thr_v = [tbuf[pl.ds(128 * j, 16)] for j in range(4)]
    bc_v = [jnp.full((16,), jnp.float32(c)) for c in _BC]
    neg_big = jnp.full((16,), jnp.float32(-3.0e38))
    one_v = jnp.full((16,), jnp.float32(1.0))
    zero_v = jnp.full((16,), jnp.float32(0.0))

    @plsc.parallel_loop(0, _CHUNK, 1, unroll=4)
    def body(t):
        row = jnp.full((16,), t * N_FEAT, jnp.int32)
        p = []
        for s in range(4):
            g = plsc.load_gather(fbuf, [row + i_cols[s]])
            bc = bc_v[0]
            bc = jnp.where(g >= thr_v[0], bc_v[1], bc)
            bc = jnp.where(g >= thr_v[1], bc_v[2], bc)
            bc = jnp.where(g >= thr_v[2], bc_v[3], bc)
            bc = jnp.where(g >= thr_v[3], bc_v[4], bc)
            p.append(bc * m_cols[s])
        ssum = (p[0] + p[2]) + (p[1] + p[3])
        logits = ssum / cnt + bias_v
        m1 = jnp.max(logits)
        oh1 = lanes == plsc.all_reduce_ffs(logits == m1)
        l2 = jnp.where(oh1, neg_big, logits)
        m2 = jnp.max(l2)
        oh2 = lanes == plsc.all_reduce_ffs(l2 == m2)
        u = jnp.exp(jnp.full((16,), m2 - m1))
        den = one_v + u
        w1 = one_v / den
        w2 = u / den
        wv = jnp.where(oh1, w1, jnp.where(oh2, w2, zero_v))
        lbuf[pl.ds(t * N_EXPERTS, N_EXPERTS)] = logits
        wbuf[pl.ds(t * N_EXPERTS, N_EXPERTS)] = wv

    pltpu.sync_copy(wbuf, w_out.at[pl.ds(base * N_EXPERTS, _CHUNK * N_EXPERTS)])
    pltpu.sync_copy(lbuf, l_out.at[pl.ds(base * N_EXPERTS, _CHUNK * N_EXPERTS)])


def _make_sc_router():
    return pl.kernel(
        _sc_router_body,
        out_type=[
            jax.ShapeDtypeStruct((N_TOK * N_EXPERTS,), jnp.float32),
            jax.ShapeDtypeStruct((N_TOK * N_EXPERTS,), jnp.float32),
        ],
        mesh=plsc.VectorSubcoreMesh(
            core_axis_name="c", subcore_axis_name="s",
            num_cores=_NC, num_subcores=_NS),
        scratch_types=[
            pltpu.VMEM((_CHUNK * N_FEAT,), jnp.float32),
            pltpu.VMEM((_CHUNK * N_EXPERTS,), jnp.float32),
            pltpu.VMEM((_CHUNK * N_EXPERTS,), jnp.float32),
            pltpu.VMEM((N_EXPERTS * N_SEL,), jnp.float32),
            pltpu.VMEM((N_EXPERTS,), jnp.float32),
            pltpu.VMEM((N_EXPERTS * N_SEL,), jnp.int32),
            pltpu.VMEM((1024,), jnp.float32),
        ],
        compiler_params=pltpu.CompilerParams(needs_layout_passes=False),
    )


def kernel(rule_features, selected_mask, expert_bias, selected_idx):
    idx_flat = selected_idx.reshape(-1).astype(jnp.int32)
    colmask = jnp.zeros((N_FEAT,), jnp.float32).at[idx_flat].set(1.0)
    thr = _thresholds_tc(rule_features, colmask.reshape(1, N_FEAT))
    w_flat, l_flat = _make_sc_router()(
        rule_features.reshape(-1),
        selected_mask.reshape(-1).astype(jnp.float32),
        expert_bias.astype(jnp.float32),
        idx_flat,
        thr.reshape(-1),
    )
    weights = w_flat.reshape(N_TOK, N_EXPERTS)
    scaled_logits = l_flat.reshape(N_TOK, N_EXPERTS)
    return (weights, scaled_logits)


# speedup vs baseline: 2.5102x; 1.1381x over previous
import struct

import jax
import jax.numpy as jnp
from jax import lax
from jax.experimental import pallas as pl
from jax.experimental.pallas import tpu as pltpu
from jax.experimental.pallas import tpu_sc as plsc

N_TOK = 16384
N_FEAT = 64
N_EXPERTS = 16
N_SEL = 4
N_BINS = 5

_NC = 2
_NS = 16
_NW = _NC * _NS
_CHUNK = N_TOK // _NW
_SPAN = 2 * _CHUNK


def _fbits(i):
    return struct.unpack('<f', struct.pack('<I', i))[0]


_ERF_T = [_fbits(0xBF57747E), _fbits(0xBE81B6B6),
          _fbits(0x3E81B6B6), _fbits(0x3F57747F)]
_CLAMP_T = [_fbits(0x3E4CCCCD), _fbits(0x3ECCCCCD),
            _fbits(0x3F19999A), _fbits(0x3F4CCCCD)]
_BC = [_fbits(0x3DCCCCCD), _fbits(0x3E99999A), _fbits(0x3F000000),
       _fbits(0x3F333333), _fbits(0x3F666667)]


def _sc_router_body(feat, maskf, bias, idxf,
                    w_out, l_out,
                    fbuf, wbuf, lbuf, mbuf, bbuf, ibuf, colbuf, pbuf, gbuf,
                    shared):
    c = lax.axis_index("c")
    s = lax.axis_index("s")
    wid = s * _NC + c
    base = wid * _CHUNK

    pltpu.sync_copy(feat.at[pl.ds(s * _SPAN * N_FEAT, _SPAN * N_FEAT)], fbuf)
    pltpu.sync_copy(maskf, mbuf)
    pltpu.sync_copy(bias, bbuf)
    pltpu.sync_copy(idxf, ibuf)

    lanes = lax.iota(jnp.int32, 16)
    four = jnp.full((16,), 4, jnp.int32)
    m_cols = [plsc.load_gather(mbuf, [lanes * four + k]) for k in range(4)]
    i_cols = [plsc.load_gather(ibuf, [lanes * four + k]) for k in range(4)]
    bias_v = bbuf[...]
    cnt = jnp.maximum((m_cols[0] + m_cols[2]) + (m_cols[1] + m_cols[3]),
                      jnp.full((16,), jnp.float32(1.0)))
    one_v = jnp.full((16,), jnp.float32(1.0))
    zero_v = jnp.full((16,), jnp.float32(0.0))
    neg_big = jnp.full((16,), jnp.float32(-3.0e38))
    big = jnp.full((16,), jnp.float32(3.4028235e38))
    nbig = -big

    for k in range(4):
        colbuf[pl.ds(16 * k, 16)] = zero_v
    for k in range(4):
        plsc.store_scatter(colbuf, [i_cols[k]], one_v)
    cm = [colbuf[pl.ds(16 * k, 16)] > zero_v for k in range(4)]

    @plsc.parallel_loop(0, _SPAN, 1, unroll=2,
                        carry=(big, big, big, big, nbig, nbig, nbig, nbig))
    def acc(t, mm):
        mn0, mn1, mn2, mn3, mx0, mx1, mx2, mx3 = mm
        o = t * N_FEAT
        x0 = fbuf[pl.ds(o, 16)]
        x1 = fbuf[pl.ds(o + 16, 16)]
        x2 = fbuf[pl.ds(o + 32, 16)]
        x3 = fbuf[pl.ds(o + 48, 16)]
        mn0 = jnp.minimum(mn0, jnp.where(cm[0], x0, big))
        mn1 = jnp.minimum(mn1, jnp.where(cm[1], x1, big))
        mn2 = jnp.minimum(mn2, jnp.where(cm[2], x2, big))
        mn3 = jnp.minimum(mn3, jnp.where(cm[3], x3, big))
        mx0 = jnp.maximum(mx0, jnp.where(cm[0], x0, nbig))
        mx1 = jnp.maximum(mx1, jnp.where(cm[1], x1, nbig))
        mx2 = jnp.maximum(mx2, jnp.where(cm[2], x2, nbig))
        mx3 = jnp.maximum(mx3, jnp.where(cm[3], x3, nbig))
        return (mn0, mn1, mn2, mn3, mx0, mx1, mx2, mx3)

    mn0, mn1, mn2, mn3, mx0, mx1, mx2, mx3 = acc
    mnv = jnp.minimum(jnp.minimum(mn0, mn1), jnp.minimum(mn2, mn3))
    mxv = jnp.maximum(jnp.maximum(mx0, mx1), jnp.maximum(mx2, mx3))
    pbuf[pl.ds(0, 16)] = mnv
    pbuf[pl.ds(16, 16)] = mxv
    pltpu.sync_copy(pbuf, shared.at[pl.ds(s * 32, 32)])
    plsc.subcore_barrier()
    pltpu.sync_copy(shared, gbuf)
    am = gbuf[pl.ds(0, 16)]
    ax = gbuf[pl.ds(16, 16)]
    for i in range(1, 16):
        am = jnp.minimum(am, gbuf[pl.ds(i * 32, 16)])
        ax = jnp.maximum(ax, gbuf[pl.ds(i * 32 + 16, 16)])
    lo = jnp.min(am)
    hi = jnp.max(ax)
    already_ratio = jnp.logical_and(lo >= -1e-06, hi <= 1.0 + 1e-06)
    flagv = jnp.full((16,), already_ratio)
    thr_v = [jnp.where(flagv,
                       jnp.full((16,), jnp.float32(ct)),
                       jnp.full((16,), jnp.float32(et)))
             for ct, et in zip(_CLAMP_T, _ERF_T)]
    bc_v = [jnp.full((16,), jnp.float32(v)) for v in _BC]

    row0 = (c * _CHUNK) * N_FEAT

    @plsc.parallel_loop(0, _CHUNK, 1, unroll=4)
    def body(t):
        row = jnp.full((16,), row0 + t * N_FEAT, jnp.int32)
        p = []
        for k in range(4):
            g = plsc.load_gather(fbuf, [row + i_cols[k]])
            bc = bc_v[0]
            bc = jnp.where(g >= thr_v[0], bc_v[1], bc)
            bc = jnp.where(g >= thr_v[1], bc_v[2], bc)
            bc = jnp.where(g >= thr_v[2], bc_v[3], bc)
            bc = jnp.where(g >= thr_v[3], bc_v[4], bc)
            p.append(bc * m_cols[k])
        ssum = (p[0] + p[2]) + (p[1] + p[3])
        logits = ssum / cnt + bias_v
        m1 = jnp.max(logits)
        oh1 = lanes == plsc.all_reduce_ffs(logits == m1)
        l2 = jnp.where(oh1, neg_big, logits)
        m2 = jnp.max(l2)
        oh2 = lanes == plsc.all_reduce_ffs(l2 == m2)
        u = jnp.exp(jnp.full((16,), m2 - m1))
        den = one_v + u
        w1 = one_v / den
        w2 = u / den
        wv = jnp.where(oh1, w1, jnp.where(oh2, w2, zero_v))
        lbuf[pl.ds(t * N_EXPERTS, N_EXPERTS)] = logits
        wbuf[pl.ds(t * N_EXPERTS, N_EXPERTS)] = wv

    pltpu.sync_copy(wbuf, w_out.at[pl.ds(base * N_EXPERTS, _CHUNK * N_EXPERTS)])
    pltpu.sync_copy(lbuf, l_out.at[pl.ds(base * N_EXPERTS, _CHUNK * N_EXPERTS)])


def _make_sc_router():
    return pl.kernel(
        _sc_router_body,
        out_type=[
            jax.ShapeDtypeStruct((N_TOK * N_EXPERTS,), jnp.float32),
            jax.ShapeDtypeStruct((N_TOK * N_EXPERTS,), jnp.float32),
        ],
        mesh=plsc.VectorSubcoreMesh(
            core_axis_name="c", subcore_axis_name="s",
            num_cores=_NC, num_subcores=_NS),
        scratch_types=[
            pltpu.VMEM((_SPAN * N_FEAT,), jnp.float32),
            pltpu.VMEM((_CHUNK * N_EXPERTS,), jnp.float32),
            pltpu.VMEM((_CHUNK * N_EXPERTS,), jnp.float32),
            pltpu.VMEM((N_EXPERTS * N_SEL,), jnp.float32),
            pltpu.VMEM((N_EXPERTS,), jnp.float32),
            pltpu.VMEM((N_EXPERTS * N_SEL,), jnp.int32),
            pltpu.VMEM((N_FEAT,), jnp.float32),
            pltpu.VMEM((32,), jnp.float32),
            pltpu.VMEM((_NS * 32,), jnp.float32),
            pltpu.VMEM_SHARED((_NS * 32,), jnp.float32),
        ],
        compiler_params=pltpu.CompilerParams(needs_layout_passes=False),
    )


def kernel(rule_features, selected_mask, expert_bias, selected_idx):
    idx_flat = selected_idx.reshape(-1).astype(jnp.int32)
    w_flat, l_flat = _make_sc_router()(
        rule_features.reshape(-1),
        selected_mask.reshape(-1).astype(jnp.float32),
        expert_bias.astype(jnp.float32),
        idx_flat,
    )
    weights = w_flat.reshape(N_TOK, N_EXPERTS)
    scaled_logits = l_flat.reshape(N_TOK, N_EXPERTS)
    return (weights, scaled_logits)


# speedup vs baseline: 2.5384x; 1.0112x over previous
import struct

import jax
import jax.numpy as jnp
from jax import lax
from jax.experimental import pallas as pl
from jax.experimental.pallas import tpu as pltpu
from jax.experimental.pallas import tpu_sc as plsc

N_TOK = 16384
N_FEAT = 64
N_EXPERTS = 16
N_SEL = 4
N_BINS = 5

_NC = 2
_NS = 16
_NW = _NC * _NS
_CHUNK = N_TOK // _NW
_SPAN = 2 * _CHUNK


def _fbits(i):
    return struct.unpack('<f', struct.pack('<I', i))[0]


_ERF_T = [_fbits(0xBF57747E), _fbits(0xBE81B6B6),
          _fbits(0x3E81B6B6), _fbits(0x3F57747F)]
_CLAMP_T = [_fbits(0x3E4CCCCD), _fbits(0x3ECCCCCD),
            _fbits(0x3F19999A), _fbits(0x3F4CCCCD)]
_BC = [_fbits(0x3DCCCCCD), _fbits(0x3E99999A), _fbits(0x3F000000),
       _fbits(0x3F333333), _fbits(0x3F666667)]


def _sc_router_body(feat, maskf, bias, idxf,
                    w_out, l_out,
                    fbuf, wbuf, lbuf, mbuf, bbuf, ibuf, colbuf, pbuf, gbuf,
                    shared):
    c = lax.axis_index("c")
    s = lax.axis_index("s")
    wid = s * _NC + c
    base = wid * _CHUNK

    pltpu.sync_copy(feat.at[pl.ds(s * _SPAN, _SPAN)], fbuf)
    pltpu.sync_copy(maskf, mbuf)
    pltpu.sync_copy(bias, bbuf)
    pltpu.sync_copy(idxf, ibuf)

    lanes = lax.iota(jnp.int32, 16)
    four = jnp.full((16,), 4, jnp.int32)
    m_cols = [plsc.load_gather(mbuf, [lanes * four + k]) for k in range(4)]
    i_cols = [plsc.load_gather(ibuf, [lanes * four + k]) for k in range(4)]
    bias_v = bbuf[...]
    cnt = jnp.maximum((m_cols[0] + m_cols[2]) + (m_cols[1] + m_cols[3]),
                      jnp.full((16,), jnp.float32(1.0)))
    one_v = jnp.full((16,), jnp.float32(1.0))
    zero_v = jnp.full((16,), jnp.float32(0.0))
    neg_big = jnp.full((16,), jnp.float32(-3.0e38))
    big = jnp.full((16,), jnp.float32(3.4028235e38))
    nbig = -big

    for k in range(4):
        colbuf[pl.ds(16 * k, 16)] = zero_v
    for k in range(4):
        plsc.store_scatter(colbuf, [i_cols[k]], one_v)
    cm = [colbuf[pl.ds(16 * k, 16)] > zero_v for k in range(4)]

    @plsc.parallel_loop(0, _SPAN, 1, unroll=2,
                        carry=(big, big, big, big, nbig, nbig, nbig, nbig))
    def acc(t, mm):
        mn0, mn1, mn2, mn3, mx0, mx1, mx2, mx3 = mm
        x0 = fbuf[t, pl.ds(0, 16)]
        x1 = fbuf[t, pl.ds(16, 16)]
        x2 = fbuf[t, pl.ds(32, 16)]
        x3 = fbuf[t, pl.ds(48, 16)]
        mn0 = jnp.minimum(mn0, jnp.where(cm[0], x0, big))
        mn1 = jnp.minimum(mn1, jnp.where(cm[1], x1, big))
        mn2 = jnp.minimum(mn2, jnp.where(cm[2], x2, big))
        mn3 = jnp.minimum(mn3, jnp.where(cm[3], x3, big))
        mx0 = jnp.maximum(mx0, jnp.where(cm[0], x0, nbig))
        mx1 = jnp.maximum(mx1, jnp.where(cm[1], x1, nbig))
        mx2 = jnp.maximum(mx2, jnp.where(cm[2], x2, nbig))
        mx3 = jnp.maximum(mx3, jnp.where(cm[3], x3, nbig))
        return (mn0, mn1, mn2, mn3, mx0, mx1, mx2, mx3)

    mn0, mn1, mn2, mn3, mx0, mx1, mx2, mx3 = acc
    mnv = jnp.minimum(jnp.minimum(mn0, mn1), jnp.minimum(mn2, mn3))
    mxv = jnp.maximum(jnp.maximum(mx0, mx1), jnp.maximum(mx2, mx3))
    pbuf[pl.ds(0, 16)] = mnv
    pbuf[pl.ds(16, 16)] = mxv
    pltpu.sync_copy(pbuf, shared.at[pl.ds(s * 32, 32)])
    plsc.subcore_barrier()
    pltpu.sync_copy(shared, gbuf)
    am = gbuf[pl.ds(0, 16)]
    ax = gbuf[pl.ds(16, 16)]
    for i in range(1, 16):
        am = jnp.minimum(am, gbuf[pl.ds(i * 32, 16)])
        ax = jnp.maximum(ax, gbuf[pl.ds(i * 32 + 16, 16)])
    lo = jnp.min(am)
    hi = jnp.max(ax)
    already_ratio = jnp.logical_and(lo >= -1e-06, hi <= 1.0 + 1e-06)
    flagv = jnp.full((16,), already_ratio)
    thr_v = [jnp.where(flagv,
                       jnp.full((16,), jnp.float32(ct)),
                       jnp.full((16,), jnp.float32(et)))
             for ct, et in zip(_CLAMP_T, _ERF_T)]
    bc_v = [jnp.full((16,), jnp.float32(v)) for v in _BC]

    row0 = c * _CHUNK

    @plsc.parallel_loop(0, _CHUNK, 1, unroll=4)
    def body(t):
        row = jnp.full((16,), row0 + t, jnp.int32)
        p = []
        for k in range(4):
            g = plsc.load_gather(fbuf, [row, i_cols[k]])
            bc = bc_v[0]
            bc = jnp.where(g >= thr_v[0], bc_v[1], bc)
            bc = jnp.where(g >= thr_v[1], bc_v[2], bc)
            bc = jnp.where(g >= thr_v[2], bc_v[3], bc)
            bc = jnp.where(g >= thr_v[3], bc_v[4], bc)
            p.append(bc * m_cols[k])
        ssum = (p[0] + p[2]) + (p[1] + p[3])
        logits = ssum / cnt + bias_v
        m1 = jnp.max(logits)
        oh1 = lanes == plsc.all_reduce_ffs(logits == m1)
        l2 = jnp.where(oh1, neg_big, logits)
        m2 = jnp.max(l2)
        oh2 = lanes == plsc.all_reduce_ffs(l2 == m2)
        u = jnp.exp(jnp.full((16,), m2 - m1))
        den = one_v + u
        w1 = one_v / den
        w2 = u / den
        wv = jnp.where(oh1, w1, jnp.where(oh2, w2, zero_v))
        lbuf[t, :] = logits
        wbuf[t, :] = wv

    pltpu.sync_copy(wbuf, w_out.at[pl.ds(base, _CHUNK)])
    pltpu.sync_copy(lbuf, l_out.at[pl.ds(base, _CHUNK)])


def _make_sc_router():
    return pl.kernel(
        _sc_router_body,
        out_type=[
            jax.ShapeDtypeStruct((N_TOK, N_EXPERTS), jnp.float32),
            jax.ShapeDtypeStruct((N_TOK, N_EXPERTS), jnp.float32),
        ],
        mesh=plsc.VectorSubcoreMesh(
            core_axis_name="c", subcore_axis_name="s",
            num_cores=_NC, num_subcores=_NS),
        scratch_types=[
            pltpu.VMEM((_SPAN, N_FEAT), jnp.float32),
            pltpu.VMEM((_CHUNK, N_EXPERTS), jnp.float32),
            pltpu.VMEM((_CHUNK, N_EXPERTS), jnp.float32),
            pltpu.VMEM((N_EXPERTS * N_SEL,), jnp.float32),
            pltpu.VMEM((N_EXPERTS,), jnp.float32),
            pltpu.VMEM((N_EXPERTS * N_SEL,), jnp.int32),
            pltpu.VMEM((N_FEAT,), jnp.float32),
            pltpu.VMEM((32,), jnp.float32),
            pltpu.VMEM((_NS * 32,), jnp.float32),
            pltpu.VMEM_SHARED((_NS * 32,), jnp.float32),
        ],
        compiler_params=pltpu.CompilerParams(
            needs_layout_passes=False, use_tc_tiling_on_sc=False),
    )


def kernel(rule_features, selected_mask, expert_bias, selected_idx):
    weights, scaled_logits = _make_sc_router()(
        rule_features,
        selected_mask.reshape(-1).astype(jnp.float32),
        expert_bias.astype(jnp.float32),
        selected_idx.reshape(-1).astype(jnp.int32),
    )
    return (weights, scaled_logits)
